# baseline (speedup 1.0000x reference)
import functools

import jax
import jax.numpy as jnp
from jax import lax
from jax.experimental import pallas as pl
from jax.experimental.pallas import tpu as pltpu
from jax.experimental.pallas import tpu_sc as plsc

VOCAB = 100000
HIDDEN = 128
BATCH = 1024


_NC, _NS = 2, 16
_NW = _NC * _NS
_B_PER_W = BATCH // _NW


@functools.cache
def _make_sc_gather():
    mesh = plsc.VectorSubcoreMesh(core_axis_name="c", subcore_axis_name="s")

    @functools.partial(
        pl.kernel,
        mesh=mesh,
        out_type=jax.ShapeDtypeStruct((BATCH, HIDDEN), jnp.float32),
        scratch_types=[
            pltpu.VMEM((_B_PER_W,), jnp.int32),
            pltpu.VMEM((_B_PER_W, HIDDEN), jnp.float32),
            pltpu.SemaphoreType.DMA,
        ],
    )
    def sc_gather(table_hbm, idx_hbm, out_hbm, idx_v, rows_v, sem):
        wid = lax.axis_index("s") * _NC + lax.axis_index("c")
        base = wid * _B_PER_W
        pltpu.sync_copy(idx_hbm.at[pl.ds(base, _B_PER_W)], idx_v)
        pltpu.async_copy(table_hbm.at[idx_v], rows_v, sem).wait()
        pltpu.sync_copy(rows_v, out_hbm.at[pl.ds(base, _B_PER_W)])

    return sc_gather


_TILE_N = 2048


def _proj_body(hidden_ref, w_ref, b_ref, out_ref):
    acc = lax.dot_general(
        hidden_ref[...], w_ref[...],
        dimension_numbers=(((1,), (1,)), ((), ())),
        preferred_element_type=jnp.float32,
    )
    out_ref[...] = acc + b_ref[...]


def _projection(hidden, lm_w, lm_b2d):
    grid = (pl.cdiv(VOCAB, _TILE_N),)
    return pl.pallas_call(
        _proj_body,
        grid=grid,
        in_specs=[
            pl.BlockSpec((BATCH, HIDDEN), lambda j: (0, 0)),
            pl.BlockSpec((_TILE_N, HIDDEN), lambda j: (j, 0)),
            pl.BlockSpec((1, _TILE_N), lambda j: (0, j)),
        ],
        out_specs=pl.BlockSpec((BATCH, _TILE_N), lambda j: (0, j)),
        out_shape=jax.ShapeDtypeStruct((BATCH, VOCAB), jnp.float32),
    )(hidden, lm_w, lm_b2d)


def kernel(input_ids, embed_table, lm_w, lm_b):
    hidden = _make_sc_gather()(embed_table, input_ids.astype(jnp.int32))
    return _projection(hidden, lm_w, lm_b.reshape(1, VOCAB))
